# baseline (speedup 1.0000x reference)
import functools

import numpy as np
import jax
import jax.numpy as jnp
from jax import lax
from jax.experimental import pallas as pl
from jax.experimental.pallas import tpu as pltpu
from jax.experimental.pallas import tpu_sc as plsc

D_MODEL = 768
MAX_LEN = 5000

NUM_CORES = 2
NUM_SUBCORES = 16
NUM_WORKERS = NUM_CORES * NUM_SUBCORES
LANES = 16
STEP_ROWS = 16
NBUF = 4


def _pe_table(time_steps: int) -> np.ndarray:
    half_dim = D_MODEL // 2
    pe = np.zeros((D_MODEL, MAX_LEN), dtype=np.float64)
    pos = np.arange(MAX_LEN)
    freq = 10000 ** (2 * np.arange(half_dim) / D_MODEL)
    pos_freq = pos.reshape((1, -1)) / freq.reshape((-1, 1))
    pe[:half_dim, :] = np.sin(pos_freq)
    pe[half_dim:, :] = np.cos(pos_freq)
    return pe.T[:time_steps].astype(np.float32)


@functools.partial(jax.jit, static_argnames=("batch", "seq_len"))
def _sc_embed(x_flat, pe, table, *, batch, seq_len):
    rows_total = batch * seq_len
    chunk = seq_len // NUM_WORKERS
    slices = chunk // STEP_ROWS
    steps = slices * batch
    depth = NBUF - 1
    scale = float(np.sqrt(np.float32(D_MODEL)))
    vregs_per_row = D_MODEL // LANES

    mesh = plsc.VectorSubcoreMesh(
        core_axis_name="c", subcore_axis_name="s")

    @functools.partial(
        pl.kernel,
        out_type=jax.ShapeDtypeStruct((rows_total, D_MODEL), jnp.float32),
        mesh=mesh,
        scratch_types=(
            [pltpu.VMEM((steps, STEP_ROWS), jnp.int32)]
            + [pltpu.VMEM((STEP_ROWS, D_MODEL), jnp.float32)] * slices
            + [pltpu.VMEM((STEP_ROWS, D_MODEL), jnp.float32)] * NBUF
            + [pltpu.SemaphoreType.DMA] * (slices + 2 * NBUF + 1)
        ),
    )
    def k(x_hbm, pe_hbm, table_hbm, out_hbm, *rest):
        idx_all = rest[0]
        pe_v = rest[1:1 + slices]
        rows_v = rest[1 + slices:1 + slices + NBUF]
        sems = rest[1 + slices + NBUF:]
        sem_pe = sems[:slices]
        sem_g = sems[slices:slices + NBUF]
        sem_s = sems[slices + NBUF:slices + 2 * NBUF]
        gsx = sems[-1]

        wid = lax.axis_index("s") * NUM_CORES + lax.axis_index("c")
        t0 = wid * chunk

        def stage_idx(s):
            h, b = divmod(s, batch)
            return pltpu.async_copy(
                x_hbm.at[pl.ds(b * seq_len + t0 + h * STEP_ROWS,
                               STEP_ROWS)],
                idx_all.at[s], gsx)

        first_idx = [stage_idx(s) for s in range(min(depth, steps))]
        for cp in first_idx:
            cp.wait()

        def start_gather(s, buf):
            return pltpu.async_copy(
                table_hbm.at[idx_all.at[s]], rows_v[buf], sem_g[buf])

        def compute(buf, h):
            def row_body(r, _):
                for c in range(vregs_per_row):
                    sl = pl.ds(c * LANES, LANES)
                    rows_v[buf][r, sl] = (
                        rows_v[buf][r, sl] * scale + pe_v[h][r, sl])
                return 0
            lax.fori_loop(0, STEP_ROWS, row_body, 0)

        gather = [None] * NBUF
        stores = [None] * NBUF
        for s0 in range(min(depth, steps)):
            gather[s0 % NBUF] = start_gather(s0, s0 % NBUF)
        pe_cp = [
            pltpu.async_copy(
                pe_hbm.at[pl.ds(t0 + h * STEP_ROWS, STEP_ROWS)],
                pe_v[h], sem_pe[h])
            for h in range(slices)
        ]
        rest_idx = [stage_idx(s) for s in range(depth, steps)]
        for s in range(steps):
            buf = s % NBUF
            h, b = divmod(s, batch)
            if s + depth < steps:
                nbuf = (s + depth) % NBUF
                if stores[nbuf] is not None:
                    stores[nbuf].wait()
                    stores[nbuf] = None
                rest_idx[s].wait()
                gather[nbuf] = start_gather(s + depth, nbuf)
            gather[buf].wait()
            if b == 0:
                pe_cp[h].wait()
            compute(buf, h)
            stores[buf] = pltpu.async_copy(
                rows_v[buf],
                out_hbm.at[pl.ds(b * seq_len + t0 + h * STEP_ROWS,
                                 STEP_ROWS)],
                sem_s[buf])
        for st in stores:
            if st is not None:
                st.wait()

    return k(x_flat, pe, table)


def kernel(x, table):
    batch, seq_len = x.shape
    pe = jnp.asarray(_pe_table(seq_len))
    out = _sc_embed(x.reshape(-1), pe, table, batch=batch, seq_len=seq_len)
    return out.reshape(batch, seq_len, D_MODEL)


# speedup vs baseline: 1.2096x; 1.2096x over previous
import functools

import numpy as np
import jax
import jax.numpy as jnp
from jax import lax
from jax.experimental import pallas as pl
from jax.experimental.pallas import tpu as pltpu
from jax.experimental.pallas import tpu_sc as plsc

D_MODEL = 768
MAX_LEN = 5000

NUM_CORES = 2
NUM_SUBCORES = 16
NUM_WORKERS = NUM_CORES * NUM_SUBCORES
LANES = 16
STEP_ROWS = 32
NBUF = 3


def _pe_table(time_steps: int) -> np.ndarray:
    half_dim = D_MODEL // 2
    pe = np.zeros((D_MODEL, MAX_LEN), dtype=np.float64)
    pos = np.arange(MAX_LEN)
    freq = 10000 ** (2 * np.arange(half_dim) / D_MODEL)
    pos_freq = pos.reshape((1, -1)) / freq.reshape((-1, 1))
    pe[:half_dim, :] = np.sin(pos_freq)
    pe[half_dim:, :] = np.cos(pos_freq)
    return pe.T[:time_steps].astype(np.float32)


@functools.partial(jax.jit, static_argnames=("batch", "seq_len"))
def _sc_embed(x_flat, pe, table, *, batch, seq_len):
    rows_total = batch * seq_len
    chunk = seq_len // NUM_WORKERS
    slices = chunk // STEP_ROWS
    steps = slices * batch
    depth = NBUF - 1
    scale = float(np.sqrt(np.float32(D_MODEL)))
    vregs_per_row = D_MODEL // LANES

    mesh = plsc.VectorSubcoreMesh(
        core_axis_name="c", subcore_axis_name="s")

    @functools.partial(
        pl.kernel,
        out_type=jax.ShapeDtypeStruct((rows_total, D_MODEL), jnp.float32),
        mesh=mesh,
        scratch_types=(
            [pltpu.VMEM((steps, STEP_ROWS), jnp.int32)]
            + [pltpu.VMEM((STEP_ROWS, D_MODEL), jnp.float32)] * slices
            + [pltpu.VMEM((STEP_ROWS, D_MODEL), jnp.float32)] * NBUF
            + [pltpu.SemaphoreType.DMA] * (slices + 2 * NBUF + 1)
        ),
    )
    def k(x_hbm, pe_hbm, table_hbm, out_hbm, *rest):
        idx_all = rest[0]
        pe_v = rest[1:1 + slices]
        rows_v = rest[1 + slices:1 + slices + NBUF]
        sems = rest[1 + slices + NBUF:]
        sem_pe = sems[:slices]
        sem_g = sems[slices:slices + NBUF]
        sem_s = sems[slices + NBUF:slices + 2 * NBUF]
        gsx = sems[-1]

        wid = lax.axis_index("s") * NUM_CORES + lax.axis_index("c")
        t0 = wid * chunk

        def stage_idx(s):
            h, b = divmod(s, batch)
            return pltpu.async_copy(
                x_hbm.at[pl.ds(b * seq_len + t0 + h * STEP_ROWS,
                               STEP_ROWS)],
                idx_all.at[s], gsx)

        first_idx = [stage_idx(s) for s in range(min(depth, steps))]
        for cp in first_idx:
            cp.wait()

        def start_gather(s, buf):
            return pltpu.async_copy(
                table_hbm.at[idx_all.at[s]], rows_v[buf], sem_g[buf])

        def compute(buf, h):
            def row_body(r, _):
                for c in range(vregs_per_row):
                    sl = pl.ds(c * LANES, LANES)
                    rows_v[buf][r, sl] = (
                        rows_v[buf][r, sl] * scale + pe_v[h][r, sl])
                return 0
            lax.fori_loop(0, STEP_ROWS, row_body, 0)

        gather = [None] * NBUF
        stores = [None] * NBUF
        for s0 in range(min(depth, steps)):
            gather[s0 % NBUF] = start_gather(s0, s0 % NBUF)
        pe_cp = [
            pltpu.async_copy(
                pe_hbm.at[pl.ds(t0 + h * STEP_ROWS, STEP_ROWS)],
                pe_v[h], sem_pe[h])
            for h in range(slices)
        ]
        rest_idx = [stage_idx(s) for s in range(depth, steps)]
        for s in range(steps):
            buf = s % NBUF
            h, b = divmod(s, batch)
            if s + depth < steps:
                nbuf = (s + depth) % NBUF
                if stores[nbuf] is not None:
                    stores[nbuf].wait()
                    stores[nbuf] = None
                rest_idx[s].wait()
                gather[nbuf] = start_gather(s + depth, nbuf)
            gather[buf].wait()
            if b == 0:
                pe_cp[h].wait()
            compute(buf, h)
            stores[buf] = pltpu.async_copy(
                rows_v[buf],
                out_hbm.at[pl.ds(b * seq_len + t0 + h * STEP_ROWS,
                                 STEP_ROWS)],
                sem_s[buf])
        for st in stores:
            if st is not None:
                st.wait()

    return k(x_flat, pe, table)


def kernel(x, table):
    batch, seq_len = x.shape
    pe = jnp.asarray(_pe_table(seq_len))
    out = _sc_embed(x.reshape(-1), pe, table, batch=batch, seq_len=seq_len)
    return out.reshape(batch, seq_len, D_MODEL)
